# baseline (speedup 1.0000x reference)
import functools

import jax
import jax.numpy as jnp
import numpy as np
from jax.experimental import pallas as pl
from jax.experimental.pallas import tpu as pltpu

_SRC_INDEX = (
    (0, 1, 2, 3, 4, 5, 6, 7, 8, 11, 13, 14, 15, 16, 17, 18, 19, 20, 21,
     23, 25, 26, 33, 34, 35),
    (0, 1, 2, 3, 4, 5, 6, 7, 8, 9, 11, 12, 13, 15, 20, 22, 24, 25, 27, 34),
    (0, 1, 4, 5, 6, 7, 8, 10, 19, 29, 30, 31, 32),
)
_LOGIT_SCALE = 1.0 / 0.07
_TILE_N = 16384
_NEG_BIAS = -30000.0
_NUM_OH = 8


def _round_up(x, m):
    return ((x + m - 1) // m) * m


def _fused_kernel(feat_ref, ds_ref, lab_ref, w_ref, b_ref, emb_ref, out_ref,
                  *, k_offs, num_ds):
    x = jnp.dot(w_ref[...], feat_ref[...].astype(jnp.bfloat16),
                preferred_element_type=jnp.float32)
    x = x + b_ref[...]
    inv_norm = jax.lax.rsqrt(jnp.sum(x * x, axis=0, keepdims=True) + 1e-12)
    feat_n = (x * inv_norm).astype(jnp.bfloat16)

    tn = feat_n.shape[1]
    ds = ds_ref[0]
    lab = lab_ref[0]

    row8 = jax.lax.broadcasted_iota(jnp.int32, (_NUM_OH, tn), 0)
    oh = row8 == ds

    lhs = jnp.concatenate([feat_n, oh.astype(jnp.bfloat16)], axis=0)
    ml = jnp.dot(emb_ref[...], lhs,
                 preferred_element_type=jnp.float32)

    s = jnp.sum(jnp.exp(ml), axis=0, keepdims=True)
    lse = jnp.log(s)

    lo = jnp.where(ds == 1, k_offs[1],
                   jnp.where(ds == 2, k_offs[2], k_offs[0]))
    tgt = lo + lab
    row64 = jax.lax.broadcasted_iota(jnp.int32, (ml.shape[0], tn), 0)
    picked = jnp.sum(jnp.where(row64 == tgt, ml, 0.0), axis=0,
                     keepdims=True)
    ce = lse - picked

    contrib = jnp.where(oh, ce, 0.0) + (row8 == ds + num_ds + 1).astype(
        jnp.float32)
    red = jnp.sum(contrib, axis=1, keepdims=True)
    out_ref[...] = jnp.broadcast_to(red, (_NUM_OH, 128)).reshape(
        out_ref.shape)


def _build_tables(class_embedding, proj_w, proj_b):
    c_in, d = proj_w.shape
    num_classes = class_embedding.shape[0]
    k_offs = []
    off = 0
    for s in _SRC_INDEX:
        k_offs.append(off)
        off += len(s)
    k_rows = _round_up(off, 8)

    w_t = proj_w.astype(jnp.bfloat16).T
    b_t = proj_b.astype(jnp.float32).reshape(d, 1)

    p_ls = np.zeros((k_rows, num_classes), np.float32)
    bias = np.zeros((k_rows, _NUM_OH), np.float32)
    bias[:, :len(_SRC_INDEX)] = _NEG_BIAS
    for di, idx in enumerate(_SRC_INDEX):
        for j, cls in enumerate(idx):
            p_ls[k_offs[di] + j, cls] = _LOGIT_SCALE
            bias[k_offs[di] + j, di] = 0.0
    emb_core = jnp.dot(jnp.asarray(p_ls), class_embedding.astype(jnp.float32),
                       precision=jax.lax.Precision.HIGHEST)
    emb_t = jnp.concatenate([emb_core, jnp.asarray(bias)],
                            axis=1).astype(jnp.bfloat16)
    return w_t, b_t, emb_t, tuple(k_offs), k_rows


def kernel(feat, proj_w, proj_b, class_embedding, dataset_idx, segment):
    w_t, b_t, emb_t, k_offs, k_rows = _build_tables(
        class_embedding, proj_w, proj_b)
    num_ds = len(_SRC_INDEX)

    n, c_in = feat.shape
    d = w_t.shape[0]

    tn = min(_TILE_N, _round_up(n, 128))
    n_pad = _round_up(n, tn)
    n_tiles = n_pad // tn

    dataset_idx = dataset_idx.astype(jnp.int32)
    segment = segment.astype(jnp.int32)
    feat_t = feat.T
    if n_pad != n:
        feat_t = jnp.pad(feat_t, ((0, 0), (0, n_pad - n)))
        dataset_idx = jnp.pad(dataset_idx, (0, n_pad - n), constant_values=-1)
        segment = jnp.pad(segment, (0, n_pad - n))

    kernel_fn = functools.partial(_fused_kernel, k_offs=k_offs,
                                  num_ds=num_ds)
    partials = pl.pallas_call(
        kernel_fn,
        out_shape=jax.ShapeDtypeStruct((n_tiles, _NUM_OH, 128), jnp.float32),
        grid=(n_tiles,),
        in_specs=[
            pl.BlockSpec((c_in, tn), lambda i: (0, i)),
            pl.BlockSpec((1, 1, tn), lambda i: (i, 0, 0)),
            pl.BlockSpec((1, 1, tn), lambda i: (i, 0, 0)),
            pl.BlockSpec((d, c_in), lambda i: (0, 0)),
            pl.BlockSpec((d, 1), lambda i: (0, 0)),
            pl.BlockSpec((k_rows, d + _NUM_OH), lambda i: (0, 0)),
        ],
        out_specs=pl.BlockSpec((1, _NUM_OH, 128), lambda i: (i, 0, 0)),
        compiler_params=pltpu.CompilerParams(
            dimension_semantics=("parallel",)),
    )(feat_t, dataset_idx.reshape(n_tiles, 1, tn),
      segment.reshape(n_tiles, 1, tn), w_t, b_t, emb_t)

    totals = jnp.sum(partials[:, :, 0], axis=0)
    ce_sums = totals[:num_ds]
    cnts = totals[num_ds + 1:2 * num_ds + 1]
    present = cnts > 0
    per_ds = jnp.where(present, ce_sums / jnp.maximum(cnts, 1.0), 0.0)
    num_present = jnp.maximum(jnp.sum(present.astype(jnp.float32)), 1.0)
    loss = jnp.sum(per_ds) / num_present
    return dict(loss=loss)


# speedup vs baseline: 1.0390x; 1.0390x over previous
import functools

import jax
import jax.numpy as jnp
import numpy as np
from jax.experimental import pallas as pl
from jax.experimental.pallas import tpu as pltpu

_SRC_INDEX = (
    (0, 1, 2, 3, 4, 5, 6, 7, 8, 11, 13, 14, 15, 16, 17, 18, 19, 20, 21,
     23, 25, 26, 33, 34, 35),
    (0, 1, 2, 3, 4, 5, 6, 7, 8, 9, 11, 12, 13, 15, 20, 22, 24, 25, 27, 34),
    (0, 1, 4, 5, 6, 7, 8, 10, 19, 29, 30, 31, 32),
)
_LOGIT_SCALE = 1.0 / 0.07
_TILE_N = 16384
_NEG_BIAS = -30000.0
_NUM_OH = 8


def _round_up(x, m):
    return ((x + m - 1) // m) * m


def _fused_kernel(feat_ref, ds_ref, lab_ref, w_ref, b_ref, emb_ref, out_ref,
                  *, k_offs, num_ds):
    x = jnp.dot(w_ref[...].astype(jnp.bfloat16),
                feat_ref[...].astype(jnp.bfloat16),
                preferred_element_type=jnp.float32)
    x = x + b_ref[...]
    inv_norm = jax.lax.rsqrt(jnp.sum(x * x, axis=0, keepdims=True) + 1e-12)
    feat_n = (x * inv_norm).astype(jnp.bfloat16)

    tn = feat_n.shape[1]
    ds = ds_ref[0]
    lab = lab_ref[0]

    row8 = jax.lax.broadcasted_iota(jnp.int32, (_NUM_OH, tn), 0)
    oh = row8 == ds

    lhs = jnp.concatenate([feat_n, oh.astype(jnp.bfloat16)], axis=0)
    ml = jnp.dot(emb_ref[...], lhs,
                 preferred_element_type=jnp.float32)

    s = jnp.sum(jnp.exp(ml), axis=0, keepdims=True)
    lse = jnp.log(s)

    lo = jnp.where(ds == 1, k_offs[1],
                   jnp.where(ds == 2, k_offs[2], k_offs[0]))
    tgt = lo + lab
    row64 = jax.lax.broadcasted_iota(jnp.int32, (ml.shape[0], tn), 0)
    picked = jnp.sum(jnp.where(row64 == tgt, ml, 0.0), axis=0,
                     keepdims=True)
    ce = lse - picked

    contrib = jnp.where(oh, ce, 0.0) + (row8 == ds + num_ds + 1).astype(
        jnp.float32)
    red = jnp.sum(contrib, axis=1, keepdims=True)
    out_ref[...] = jnp.broadcast_to(red, (_NUM_OH, 128)).reshape(
        out_ref.shape)


def _build_tables(class_embedding, proj_w, proj_b):
    c_in, d = proj_w.shape
    num_classes = class_embedding.shape[0]
    k_offs = []
    off = 0
    for s in _SRC_INDEX:
        k_offs.append(off)
        off += len(s)
    k_rows = _round_up(off, 8)

    w_t = proj_w.T
    b_t = proj_b.astype(jnp.float32).reshape(d, 1)

    p_ls = np.zeros((k_rows, num_classes), np.float32)
    bias = np.zeros((k_rows, _NUM_OH), np.float32)
    bias[:, :len(_SRC_INDEX)] = _NEG_BIAS
    for di, idx in enumerate(_SRC_INDEX):
        for j, cls in enumerate(idx):
            p_ls[k_offs[di] + j, cls] = _LOGIT_SCALE
            bias[k_offs[di] + j, di] = 0.0
    emb_core = jnp.dot(jnp.asarray(p_ls), class_embedding.astype(jnp.float32),
                       precision=jax.lax.Precision.HIGHEST)
    emb_t = jnp.concatenate([emb_core, jnp.asarray(bias)],
                            axis=1).astype(jnp.bfloat16)
    return w_t, b_t, emb_t, tuple(k_offs), k_rows


def kernel(feat, proj_w, proj_b, class_embedding, dataset_idx, segment):
    w_t, b_t, emb_t, k_offs, k_rows = _build_tables(
        class_embedding, proj_w, proj_b)
    num_ds = len(_SRC_INDEX)

    n, c_in = feat.shape
    d = w_t.shape[0]

    tn = min(_TILE_N, _round_up(n, 128))
    n_pad = _round_up(n, tn)
    n_tiles = n_pad // tn

    dataset_idx = dataset_idx.astype(jnp.int32)
    segment = segment.astype(jnp.int32)
    feat_t = feat.T
    if n_pad != n:
        feat_t = jnp.pad(feat_t, ((0, 0), (0, n_pad - n)))
        dataset_idx = jnp.pad(dataset_idx, (0, n_pad - n), constant_values=-1)
        segment = jnp.pad(segment, (0, n_pad - n))

    kernel_fn = functools.partial(_fused_kernel, k_offs=k_offs,
                                  num_ds=num_ds)
    partials = pl.pallas_call(
        kernel_fn,
        out_shape=jax.ShapeDtypeStruct((n_tiles, _NUM_OH, 128), jnp.float32),
        grid=(n_tiles,),
        in_specs=[
            pl.BlockSpec((c_in, tn), lambda i: (0, i)),
            pl.BlockSpec((1, 1, tn), lambda i: (i, 0, 0)),
            pl.BlockSpec((1, 1, tn), lambda i: (i, 0, 0)),
            pl.BlockSpec((d, c_in), lambda i: (0, 0)),
            pl.BlockSpec((d, 1), lambda i: (0, 0)),
            pl.BlockSpec((k_rows, d + _NUM_OH), lambda i: (0, 0)),
        ],
        out_specs=pl.BlockSpec((1, _NUM_OH, 128), lambda i: (i, 0, 0)),
        compiler_params=pltpu.CompilerParams(
            dimension_semantics=("parallel",)),
    )(feat_t, dataset_idx.reshape(n_tiles, 1, tn),
      segment.reshape(n_tiles, 1, tn), w_t, b_t, emb_t)

    totals = jnp.sum(partials[:, :, 0], axis=0)
    ce_sums = totals[:num_ds]
    cnts = totals[num_ds + 1:2 * num_ds + 1]
    present = cnts > 0
    per_ds = jnp.where(present, ce_sums / jnp.maximum(cnts, 1.0), 0.0)
    num_present = jnp.maximum(jnp.sum(present.astype(jnp.float32)), 1.0)
    loss = jnp.sum(per_ds) / num_present
    return dict(loss=loss)
